# Initial kernel scaffold; baseline (speedup 1.0000x reference)
import jax
import jax.numpy as jnp
from jax.experimental import pallas as pl


def kernel(tokens, tables, proj_W, proj_b):
    raise NotImplementedError("write your pallas kernel here")


# speedup vs baseline: 4.7191x; 4.7191x over previous
import functools

import jax
import jax.numpy as jnp
from jax import lax
from jax.experimental import pallas as pl
from jax.experimental.pallas import tpu as pltpu
from jax.experimental.pallas import tpu_sc as plsc

NUM_FIELDS = 8
VOCAB = 100000
EMB = 64
PROJ = 512
DTOT = NUM_FIELDS * EMB

NC, NS = 2, 16
NW = NC * NS
CHUNK = 128
NBUF = 2


@functools.lru_cache(maxsize=None)
def _make_gather(n_rows: int):
    rows_per_w = n_rows // NW
    chunks_per_w = rows_per_w // CHUNK
    mesh = plsc.VectorSubcoreMesh(core_axis_name="c", subcore_axis_name="s")

    assert chunks_per_w % NBUF == 0
    n_groups = chunks_per_w // NBUF

    @functools.partial(
        pl.kernel,
        mesh=mesh,
        compiler_params=pltpu.CompilerParams(use_tc_tiling_on_sc=False),
        out_type=jax.ShapeDtypeStruct((n_rows, EMB), jnp.float32),
        scratch_types=[
            pltpu.VMEM((chunks_per_w, CHUNK), jnp.int32),
            pltpu.VMEM((CHUNK, EMB), jnp.float32),
            pltpu.VMEM((CHUNK, EMB), jnp.float32),
            pltpu.SemaphoreType.DMA,
            pltpu.SemaphoreType.DMA,
            pltpu.SemaphoreType.DMA,
            pltpu.SemaphoreType.DMA,
        ],
    )
    def gather(table_hbm, idx_hbm, out_hbm, idx_v, buf0, buf1, g0, g1, w0, w1):
        wid = lax.axis_index("s") * NC + lax.axis_index("c")
        cbase = wid * chunks_per_w
        rbase = wid * rows_per_w
        bufs, gsems, wsems = [buf0, buf1], [g0, g1], [w0, w1]
        pltpu.sync_copy(idx_hbm.at[pl.ds(cbase, chunks_per_w)], idx_v)

        def gather_cp(j, b):
            return pltpu.make_async_copy(
                table_hbm.at[idx_v.at[j]], bufs[b], gsems[b]
            )

        def wb_cp(j, b):
            return pltpu.make_async_copy(
                bufs[b], out_hbm.at[pl.ds(rbase + j * CHUNK, CHUNK)], wsems[b]
            )

        for b in range(NBUF):
            gather_cp(b, b).start()

        def group(g, _):
            for b in range(NBUF):
                j = g * NBUF + b
                gather_cp(j, b).wait()
                wb_cp(j, b).start()

                @pl.when(g < n_groups - 1)
                def _():
                    wb_cp(j, b).wait()
                    gather_cp(j + NBUF, b).start()
            return _

        lax.fori_loop(0, n_groups, group, None)

        for b in range(NBUF):
            wb_cp(chunks_per_w - NBUF + b, b).wait()

    return gather


def _mm_body(x_ref, w_ref, b_ref, o_ref):
    o_ref[...] = (
        jnp.dot(x_ref[...], w_ref[...], preferred_element_type=jnp.float32)
        + b_ref[...]
    )


@functools.lru_cache(maxsize=None)
def _make_matmul(n_tok: int, bm: int):
    return pl.pallas_call(
        _mm_body,
        grid=(n_tok // bm,),
        in_specs=[
            pl.BlockSpec((bm, DTOT), lambda i: (i, 0)),
            pl.BlockSpec((DTOT, PROJ), lambda i: (0, 0)),
            pl.BlockSpec((1, PROJ), lambda i: (0, 0)),
        ],
        out_specs=pl.BlockSpec((bm, PROJ), lambda i: (i, 0)),
        out_shape=jax.ShapeDtypeStruct((n_tok, PROJ), jnp.float32),
    )


def kernel(tokens, tables, proj_W, proj_b):
    B, T, F = tokens.shape
    n_tok = B * T
    n_rows = n_tok * F

    flat_tables = tables.reshape(F * VOCAB, EMB)
    offs = jnp.arange(F, dtype=jnp.int32) * VOCAB
    idx = (tokens.astype(jnp.int32) + offs).reshape(n_rows // CHUNK, CHUNK)

    gathered = _make_gather(n_rows)(flat_tables, idx)
    x = gathered.reshape(n_tok, DTOT)

    out = _make_matmul(n_tok, 2048)(x, proj_W.T, proj_b.reshape(1, PROJ))
    return out.reshape(B, T, PROJ)


# speedup vs baseline: 6.4590x; 1.3687x over previous
import functools

import jax
import jax.numpy as jnp
from jax import lax
from jax.experimental import pallas as pl
from jax.experimental.pallas import tpu as pltpu
from jax.experimental.pallas import tpu_sc as plsc

NUM_FIELDS = 8
VOCAB = 100000
EMB = 64
PROJ = 512
DTOT = NUM_FIELDS * EMB

NC, NS = 2, 16
NW = NC * NS
CHUNK = 128
NBUF = 2


@functools.lru_cache(maxsize=None)
def _make_gather(n_rows: int):
    rows_per_w = n_rows // NW
    chunks_per_w = rows_per_w // CHUNK
    mesh = plsc.VectorSubcoreMesh(core_axis_name="c", subcore_axis_name="s")

    assert chunks_per_w % NBUF == 0
    n_groups = chunks_per_w // NBUF

    @functools.partial(
        pl.kernel,
        mesh=mesh,
        compiler_params=pltpu.CompilerParams(use_tc_tiling_on_sc=False),
        out_type=jax.ShapeDtypeStruct((n_rows, EMB), jnp.float32),
        scratch_types=[
            pltpu.VMEM((chunks_per_w, CHUNK), jnp.int32),
            pltpu.VMEM((CHUNK, EMB), jnp.float32),
            pltpu.VMEM((CHUNK, EMB), jnp.float32),
            pltpu.SemaphoreType.DMA,
            pltpu.SemaphoreType.DMA,
            pltpu.SemaphoreType.DMA,
            pltpu.SemaphoreType.DMA,
        ],
    )
    def gather(table_hbm, idx_hbm, out_hbm, idx_v, buf0, buf1, g0, g1, w0, w1):
        wid = lax.axis_index("s") * NC + lax.axis_index("c")
        cbase = wid * chunks_per_w
        rbase = wid * rows_per_w
        bufs, gsems, wsems = [buf0, buf1], [g0, g1], [w0, w1]
        pltpu.sync_copy(idx_hbm.at[pl.ds(cbase, chunks_per_w)], idx_v)

        def gather_cp(j, b):
            return pltpu.make_async_copy(
                table_hbm.at[idx_v.at[j]], bufs[b], gsems[b]
            )

        def wb_cp(j, b):
            return pltpu.make_async_copy(
                bufs[b], out_hbm.at[pl.ds(rbase + j * CHUNK, CHUNK)], wsems[b]
            )

        for b in range(NBUF):
            gather_cp(b, b).start()

        def group(g, _):
            for b in range(NBUF):
                j = g * NBUF + b
                gather_cp(j, b).wait()
                wb_cp(j, b).start()

                @pl.when(g < n_groups - 1)
                def _():
                    wb_cp(j, b).wait()
                    gather_cp(j + NBUF, b).start()
            return _

        lax.fori_loop(0, n_groups, group, None)

        for b in range(NBUF):
            wb_cp(chunks_per_w - NBUF + b, b).wait()

    return gather


def _mm_body(x_ref, w_ref, b_ref, o_ref):
    bm4 = x_ref.shape[0]
    x = x_ref[...].reshape(bm4 // 4, DTOT)
    o_ref[...] = (
        jnp.dot(x, w_ref[...], preferred_element_type=jnp.float32)
        + b_ref[...]
    )


@functools.lru_cache(maxsize=None)
def _make_matmul(n_tok: int, bm: int):
    return pl.pallas_call(
        _mm_body,
        grid=(n_tok // bm,),
        in_specs=[
            pl.BlockSpec((bm * 4, 2 * EMB), lambda i: (i, 0)),
            pl.BlockSpec((DTOT, PROJ), lambda i: (0, 0)),
            pl.BlockSpec((1, PROJ), lambda i: (0, 0)),
        ],
        out_specs=pl.BlockSpec((bm, PROJ), lambda i: (i, 0)),
        out_shape=jax.ShapeDtypeStruct((n_tok, PROJ), jnp.float32),
    )


def kernel(tokens, tables, proj_W, proj_b):
    B, T, F = tokens.shape
    n_tok = B * T
    n_rows = n_tok * F

    flat_tables = tables.reshape(F * VOCAB, EMB)
    offs = jnp.arange(F, dtype=jnp.int32) * VOCAB
    idx = (tokens.astype(jnp.int32) + offs).reshape(n_rows // CHUNK, CHUNK)

    gathered = _make_gather(n_rows)(flat_tables, idx)
    x128 = gathered.reshape(n_rows // 2, 2 * EMB)

    out = _make_matmul(n_tok, 2048)(x128, proj_W.T, proj_b.reshape(1, PROJ))
    return out.reshape(B, T, PROJ)


# speedup vs baseline: 6.4853x; 1.0041x over previous
import functools

import jax
import jax.numpy as jnp
from jax import lax
from jax.experimental import pallas as pl
from jax.experimental.pallas import tpu as pltpu
from jax.experimental.pallas import tpu_sc as plsc

NUM_FIELDS = 8
VOCAB = 100000
EMB = 64
PROJ = 512
DTOT = NUM_FIELDS * EMB

NC, NS = 2, 16
NW = NC * NS
CHUNK = 128
NBUF = 2


@functools.lru_cache(maxsize=None)
def _make_gather(n_rows: int):
    rows_per_w = n_rows // NW
    chunks_per_w = rows_per_w // CHUNK
    mesh = plsc.VectorSubcoreMesh(core_axis_name="c", subcore_axis_name="s")

    assert chunks_per_w % NBUF == 0
    n_groups = chunks_per_w // NBUF

    @functools.partial(
        pl.kernel,
        mesh=mesh,
        compiler_params=pltpu.CompilerParams(use_tc_tiling_on_sc=False),
        out_type=jax.ShapeDtypeStruct((n_rows, EMB), jnp.float32),
        scratch_types=[
            pltpu.VMEM((chunks_per_w, CHUNK), jnp.int32),
            pltpu.VMEM((CHUNK, EMB), jnp.float32),
            pltpu.VMEM((CHUNK, EMB), jnp.float32),
            pltpu.SemaphoreType.DMA,
            pltpu.SemaphoreType.DMA,
            pltpu.SemaphoreType.DMA,
            pltpu.SemaphoreType.DMA,
        ],
    )
    def gather(table_hbm, idx_hbm, out_hbm, idx_v, buf0, buf1, g0, g1, w0, w1):
        wid = lax.axis_index("s") * NC + lax.axis_index("c")
        cbase = wid * chunks_per_w
        rbase = wid * rows_per_w
        bufs, gsems, wsems = [buf0, buf1], [g0, g1], [w0, w1]
        pltpu.sync_copy(idx_hbm.at[pl.ds(cbase, chunks_per_w)], idx_v)

        def gather_cp(j, b):
            return pltpu.make_async_copy(
                table_hbm.at[idx_v.at[j]], bufs[b], gsems[b]
            )

        def wb_cp(j, b):
            return pltpu.make_async_copy(
                bufs[b], out_hbm.at[pl.ds(rbase + j * CHUNK, CHUNK)], wsems[b]
            )

        for b in range(NBUF):
            gather_cp(b, b).start()

        def group(g, _):
            for b in range(NBUF):
                j = g * NBUF + b
                gather_cp(j, b).wait()
                wb_cp(j, b).start()

                @pl.when(g < n_groups - 1)
                def _():
                    wb_cp(j, b).wait()
                    gather_cp(j + NBUF, b).start()
            return _

        lax.fori_loop(0, n_groups, group, None)

        for b in range(NBUF):
            wb_cp(chunks_per_w - NBUF + b, b).wait()

    return gather


def _mm_body(x_ref, w_ref, b_ref, o_ref):
    bm4 = x_ref.shape[0]
    x = x_ref[...].reshape(bm4 // 4, DTOT)
    o_ref[...] = (
        jnp.dot(x, w_ref[...], preferred_element_type=jnp.float32)
        + b_ref[...]
    )


def _mm_body_aliased(x_ref, w_ref, b_ref, _prev_ref, o_ref):
    _mm_body(x_ref, w_ref, b_ref, o_ref)


@functools.lru_cache(maxsize=None)
def _make_matmul(n_tok: int, bm: int, slice_blocks: int, k: int):
    base = k * slice_blocks
    in_specs = [
        pl.BlockSpec((bm * 4, 2 * EMB), lambda i: (i, 0)),
        pl.BlockSpec((DTOT, PROJ), lambda i: (0, 0)),
        pl.BlockSpec((1, PROJ), lambda i: (0, 0)),
    ]
    out_spec = pl.BlockSpec((bm, PROJ), lambda i: (i + base, 0))
    if k == 0:
        return pl.pallas_call(
            _mm_body,
            grid=(slice_blocks,),
            in_specs=in_specs,
            out_specs=out_spec,
            out_shape=jax.ShapeDtypeStruct((n_tok, PROJ), jnp.float32),
        )
    return pl.pallas_call(
        _mm_body_aliased,
        grid=(slice_blocks,),
        in_specs=in_specs + [pl.BlockSpec(memory_space=pl.ANY)],
        out_specs=out_spec,
        out_shape=jax.ShapeDtypeStruct((n_tok, PROJ), jnp.float32),
        input_output_aliases={3: 0},
    )


NSLICE = 8
BM = 1600


def kernel(tokens, tables, proj_W, proj_b):
    B, T, F = tokens.shape
    n_tok = B * T
    n_rows = n_tok * F
    tok_s = n_tok // NSLICE
    rows_s = n_rows // NSLICE
    slice_blocks = tok_s // BM

    flat_tables = tables.reshape(F * VOCAB, EMB)
    offs = jnp.arange(F, dtype=jnp.int32) * VOCAB
    idx = (tokens.astype(jnp.int32) + offs).reshape(n_rows // CHUNK, CHUNK)

    wt = proj_W.T
    b2 = proj_b.reshape(1, PROJ)
    gather_fn = _make_gather(rows_s)
    ichunks = rows_s // CHUNK

    out = None
    for k in range(NSLICE):
        g = gather_fn(flat_tables, idx[k * ichunks:(k + 1) * ichunks])
        xk = g.reshape(rows_s // 2, 2 * EMB)
        mm = _make_matmul(n_tok, BM, slice_blocks, k)
        out = mm(xk, wt, b2) if k == 0 else mm(xk, wt, b2, out)
    return out.reshape(B, T, PROJ)


# speedup vs baseline: 6.4933x; 1.0012x over previous
import functools

import jax
import jax.numpy as jnp
from jax import lax
from jax.experimental import pallas as pl
from jax.experimental.pallas import tpu as pltpu
from jax.experimental.pallas import tpu_sc as plsc

NUM_FIELDS = 8
VOCAB = 100000
EMB = 64
PROJ = 512
DTOT = NUM_FIELDS * EMB

NC, NS = 2, 16
NW = NC * NS
CHUNK = 128
NBUF = 2


@functools.lru_cache(maxsize=None)
def _make_gather(n_rows: int):
    rows_per_w = n_rows // NW
    chunks_per_w = rows_per_w // CHUNK
    mesh = plsc.VectorSubcoreMesh(core_axis_name="c", subcore_axis_name="s")

    assert chunks_per_w % NBUF == 0
    n_groups = chunks_per_w // NBUF

    @functools.partial(
        pl.kernel,
        mesh=mesh,
        compiler_params=pltpu.CompilerParams(use_tc_tiling_on_sc=False),
        out_type=jax.ShapeDtypeStruct((n_rows, EMB), jnp.float32),
        scratch_types=[
            pltpu.VMEM((chunks_per_w, CHUNK), jnp.int32),
            pltpu.VMEM((CHUNK, EMB), jnp.float32),
            pltpu.VMEM((CHUNK, EMB), jnp.float32),
            pltpu.SemaphoreType.DMA,
            pltpu.SemaphoreType.DMA,
            pltpu.SemaphoreType.DMA,
            pltpu.SemaphoreType.DMA,
        ],
    )
    def gather(table_hbm, idx_hbm, out_hbm, idx_v, buf0, buf1, g0, g1, w0, w1):
        wid = lax.axis_index("s") * NC + lax.axis_index("c")
        cbase = wid * chunks_per_w
        rbase = wid * rows_per_w
        bufs, gsems, wsems = [buf0, buf1], [g0, g1], [w0, w1]
        pltpu.sync_copy(idx_hbm.at[pl.ds(cbase, chunks_per_w)], idx_v)

        def gather_cp(j, b):
            return pltpu.make_async_copy(
                table_hbm.at[idx_v.at[j]], bufs[b], gsems[b]
            )

        def wb_cp(j, b):
            return pltpu.make_async_copy(
                bufs[b], out_hbm.at[pl.ds(rbase + j * CHUNK, CHUNK)], wsems[b]
            )

        for b in range(NBUF):
            gather_cp(b, b).start()

        def group(g, _):
            for b in range(NBUF):
                j = g * NBUF + b
                gather_cp(j, b).wait()
                wb_cp(j, b).start()

                @pl.when(g < n_groups - 1)
                def _():
                    wb_cp(j, b).wait()
                    gather_cp(j + NBUF, b).start()
            return _

        lax.fori_loop(0, n_groups, group, None)

        for b in range(NBUF):
            wb_cp(chunks_per_w - NBUF + b, b).wait()

    return gather


def _mm_body(x_ref, w_ref, b_ref, o_ref):
    bm4 = x_ref.shape[0]
    x = x_ref[...].reshape(bm4 // 4, DTOT)
    o_ref[...] = (
        jnp.dot(x, w_ref[...], preferred_element_type=jnp.float32)
        + b_ref[...]
    )


def _mm_body_aliased(x_ref, w_ref, b_ref, _prev_ref, o_ref):
    _mm_body(x_ref, w_ref, b_ref, o_ref)


@functools.lru_cache(maxsize=None)
def _make_matmul(n_tok: int, bm: int, slice_blocks: int, k: int):
    base = k * slice_blocks
    in_specs = [
        pl.BlockSpec((bm * 4, 2 * EMB), lambda i: (i, 0)),
        pl.BlockSpec((DTOT, PROJ), lambda i: (0, 0)),
        pl.BlockSpec((1, PROJ), lambda i: (0, 0)),
    ]
    out_spec = pl.BlockSpec((bm, PROJ), lambda i: (i + base, 0))
    params = pltpu.CompilerParams(skip_device_barrier=True)
    if k == 0:
        return pl.pallas_call(
            _mm_body,
            grid=(slice_blocks,),
            in_specs=in_specs,
            out_specs=out_spec,
            out_shape=jax.ShapeDtypeStruct((n_tok, PROJ), jnp.float32),
            compiler_params=params,
        )
    return pl.pallas_call(
        _mm_body_aliased,
        grid=(slice_blocks,),
        in_specs=in_specs + [pl.BlockSpec(memory_space=pl.ANY)],
        out_specs=out_spec,
        out_shape=jax.ShapeDtypeStruct((n_tok, PROJ), jnp.float32),
        input_output_aliases={3: 0},
        compiler_params=params,
    )


NSLICE = 8
BM = 1600


def kernel(tokens, tables, proj_W, proj_b):
    B, T, F = tokens.shape
    n_tok = B * T
    n_rows = n_tok * F
    tok_s = n_tok // NSLICE
    rows_s = n_rows // NSLICE
    slice_blocks = tok_s // BM

    flat_tables = tables.reshape(F * VOCAB, EMB)
    offs = jnp.arange(F, dtype=jnp.int32) * VOCAB
    idx = (tokens.astype(jnp.int32) + offs).reshape(n_rows // CHUNK, CHUNK)

    wt = proj_W.T
    b2 = proj_b.reshape(1, PROJ)
    gather_fn = _make_gather(rows_s)
    ichunks = rows_s // CHUNK

    out = None
    for k in range(NSLICE):
        g = gather_fn(flat_tables, idx[k * ichunks:(k + 1) * ichunks])
        xk = g.reshape(rows_s // 2, 2 * EMB)
        mm = _make_matmul(n_tok, BM, slice_blocks, k)
        out = mm(xk, wt, b2) if k == 0 else mm(xk, wt, b2, out)
    return out.reshape(B, T, PROJ)


# speedup vs baseline: 6.5836x; 1.0139x over previous
import functools

import jax
import jax.numpy as jnp
from jax import lax
from jax.experimental import pallas as pl
from jax.experimental.pallas import tpu as pltpu
from jax.experimental.pallas import tpu_sc as plsc

NUM_FIELDS = 8
VOCAB = 100000
EMB = 64
PROJ = 512
DTOT = NUM_FIELDS * EMB

NC, NS = 2, 16
NW = NC * NS
CHUNK = 128
NBUF = 2


@functools.lru_cache(maxsize=None)
def _make_gather(n_rows: int, slice_chunk_base: int):
    rows_per_w = n_rows // NW
    chunks_per_w = rows_per_w // CHUNK
    mesh = plsc.VectorSubcoreMesh(core_axis_name="c", subcore_axis_name="s")

    assert chunks_per_w % NBUF == 0
    n_groups = chunks_per_w // NBUF

    @functools.partial(
        pl.kernel,
        mesh=mesh,
        compiler_params=pltpu.CompilerParams(use_tc_tiling_on_sc=False),
        out_type=jax.ShapeDtypeStruct((n_rows, EMB), jnp.float32),
        scratch_types=[
            pltpu.VMEM((chunks_per_w, CHUNK), jnp.int32),
            pltpu.VMEM((CHUNK, EMB), jnp.float32),
            pltpu.VMEM((CHUNK, EMB), jnp.float32),
            pltpu.SemaphoreType.DMA,
            pltpu.SemaphoreType.DMA,
            pltpu.SemaphoreType.DMA,
            pltpu.SemaphoreType.DMA,
        ],
    )
    def gather(table_hbm, idx_hbm, out_hbm, idx_v, buf0, buf1, g0, g1, w0, w1):
        wid = lax.axis_index("s") * NC + lax.axis_index("c")
        cbase = slice_chunk_base + wid * chunks_per_w
        rbase = wid * rows_per_w
        bufs, gsems, wsems = [buf0, buf1], [g0, g1], [w0, w1]
        pltpu.sync_copy(idx_hbm.at[pl.ds(cbase, chunks_per_w)], idx_v)

        def gather_cp(j, b):
            return pltpu.make_async_copy(
                table_hbm.at[idx_v.at[j]], bufs[b], gsems[b]
            )

        def wb_cp(j, b):
            return pltpu.make_async_copy(
                bufs[b], out_hbm.at[pl.ds(rbase + j * CHUNK, CHUNK)], wsems[b]
            )

        for b in range(NBUF):
            gather_cp(b, b).start()

        def group(g, _):
            for b in range(NBUF):
                j = g * NBUF + b
                gather_cp(j, b).wait()
                wb_cp(j, b).start()

                @pl.when(g < n_groups - 1)
                def _():
                    wb_cp(j, b).wait()
                    gather_cp(j + NBUF, b).start()
            return _

        lax.fori_loop(0, n_groups, group, None)

        for b in range(NBUF):
            wb_cp(chunks_per_w - NBUF + b, b).wait()

    return gather


def _mm_body(x_ref, w_ref, b_ref, o_ref):
    bm4 = x_ref.shape[0]
    x = x_ref[...].reshape(bm4 // 4, DTOT)
    o_ref[...] = (
        jnp.dot(x, w_ref[...], preferred_element_type=jnp.float32)
        + b_ref[...]
    )


def _mm_body_aliased(x_ref, w_ref, b_ref, _prev_ref, o_ref):
    _mm_body(x_ref, w_ref, b_ref, o_ref)


@functools.lru_cache(maxsize=None)
def _make_matmul(n_tok: int, bm: int, slice_blocks: int, k: int):
    base = k * slice_blocks
    in_specs = [
        pl.BlockSpec((bm * 4, 2 * EMB), lambda i: (i, 0)),
        pl.BlockSpec((DTOT, PROJ), lambda i: (0, 0)),
        pl.BlockSpec((1, PROJ), lambda i: (0, 0)),
    ]
    out_spec = pl.BlockSpec((bm, PROJ), lambda i: (i + base, 0))
    params = pltpu.CompilerParams(skip_device_barrier=True)
    if k == 0:
        return pl.pallas_call(
            _mm_body,
            grid=(slice_blocks,),
            in_specs=in_specs,
            out_specs=out_spec,
            out_shape=jax.ShapeDtypeStruct((n_tok, PROJ), jnp.float32),
            compiler_params=params,
        )
    return pl.pallas_call(
        _mm_body_aliased,
        grid=(slice_blocks,),
        in_specs=in_specs + [pl.BlockSpec(memory_space=pl.ANY)],
        out_specs=out_spec,
        out_shape=jax.ShapeDtypeStruct((n_tok, PROJ), jnp.float32),
        input_output_aliases={3: 0},
        compiler_params=params,
    )


NSLICE = 8
BM = 1600


def kernel(tokens, tables, proj_W, proj_b):
    B, T, F = tokens.shape
    n_tok = B * T
    n_rows = n_tok * F
    tok_s = n_tok // NSLICE
    rows_s = n_rows // NSLICE
    slice_blocks = tok_s // BM

    t128 = lax.optimization_barrier(tables.reshape(F * VOCAB // 2, 2 * EMB))
    flat_tables = t128.reshape(F * VOCAB, EMB)

    offs = (jnp.arange(CHUNK, dtype=jnp.int32) % F) * VOCAB
    idx = tokens.astype(jnp.int32).reshape(n_rows // CHUNK, CHUNK) + offs

    wt = proj_W.T
    b2 = proj_b.reshape(1, PROJ)
    ichunks = rows_s // CHUNK

    out = None
    for k in range(NSLICE):
        g = _make_gather(rows_s, k * ichunks)(flat_tables, idx)
        xk = g.reshape(rows_s // 2, 2 * EMB)
        mm = _make_matmul(n_tok, BM, slice_blocks, k)
        out = mm(xk, wt, b2) if k == 0 else mm(xk, wt, b2, out)
    return out.reshape(B, T, PROJ)


# speedup vs baseline: 6.9522x; 1.0560x over previous
import functools

import jax
import jax.numpy as jnp
from jax import lax
from jax.experimental import pallas as pl
from jax.experimental.pallas import tpu as pltpu
from jax.experimental.pallas import tpu_sc as plsc

NUM_FIELDS = 8
VOCAB = 100000
EMB = 64
PROJ = 512
DTOT = NUM_FIELDS * EMB

NC, NS = 2, 16
NW = NC * NS
CHUNK = 128
NBUF = 2


@functools.lru_cache(maxsize=None)
def _make_gather(n_rows: int, slice_chunk_base: int):
    rows_per_w = n_rows // NW
    chunks_per_w = rows_per_w // CHUNK
    mesh = plsc.VectorSubcoreMesh(core_axis_name="c", subcore_axis_name="s")

    assert chunks_per_w % NBUF == 0
    n_groups = chunks_per_w // NBUF

    @functools.partial(
        pl.kernel,
        mesh=mesh,
        compiler_params=pltpu.CompilerParams(use_tc_tiling_on_sc=False),
        out_type=jax.ShapeDtypeStruct((n_rows, EMB), jnp.float32),
        scratch_types=[
            pltpu.VMEM((chunks_per_w, CHUNK), jnp.int32),
            pltpu.VMEM((CHUNK, EMB), jnp.float32),
            pltpu.VMEM((CHUNK, EMB), jnp.float32),
            pltpu.SemaphoreType.DMA,
            pltpu.SemaphoreType.DMA,
            pltpu.SemaphoreType.DMA,
            pltpu.SemaphoreType.DMA,
        ],
    )
    def gather(table_hbm, idx_hbm, out_hbm, idx_v, buf0, buf1, g0, g1, w0, w1):
        wid = lax.axis_index("s") * NC + lax.axis_index("c")
        cbase = slice_chunk_base + wid * chunks_per_w
        rbase = wid * rows_per_w
        bufs, gsems, wsems = [buf0, buf1], [g0, g1], [w0, w1]
        pltpu.sync_copy(idx_hbm.at[pl.ds(cbase, chunks_per_w)], idx_v)

        def gather_cp(j, b):
            return pltpu.make_async_copy(
                table_hbm.at[idx_v.at[j]], bufs[b], gsems[b]
            )

        def wb_cp(j, b):
            return pltpu.make_async_copy(
                bufs[b], out_hbm.at[pl.ds(rbase + j * CHUNK, CHUNK)], wsems[b]
            )

        for b in range(NBUF):
            gather_cp(b, b).start()

        def group(g, _):
            for b in range(NBUF):
                j = g * NBUF + b
                gather_cp(j, b).wait()
                wb_cp(j, b).start()

                @pl.when(g < n_groups - 1)
                def _():
                    wb_cp(j, b).wait()
                    gather_cp(j + NBUF, b).start()
            return _

        lax.fori_loop(0, n_groups, group, None)

        for b in range(NBUF):
            wb_cp(chunks_per_w - NBUF + b, b).wait()

    return gather


def _tr_body(x_ref, o_ref):
    vb = x_ref.shape[2]
    x = x_ref[0]
    o_ref[0] = jnp.concatenate([x[:, : vb // 2].T, x[:, vb // 2:].T], axis=1)


@functools.lru_cache(maxsize=None)
def _make_transpose(vb: int):
    nvb = -(-VOCAB // vb)
    return pl.pallas_call(
        _tr_body,
        grid=(NUM_FIELDS, nvb),
        in_specs=[pl.BlockSpec((1, EMB, vb), lambda f, v: (f, 0, v))],
        out_specs=pl.BlockSpec((1, vb // 2, 2 * EMB), lambda f, v: (f, v, 0)),
        out_shape=jax.ShapeDtypeStruct(
            (NUM_FIELDS, VOCAB // 2, 2 * EMB), jnp.float32
        ),
    )


def _mm_body(x_ref, w_ref, b_ref, o_ref):
    bm4 = x_ref.shape[0]
    x = x_ref[...].reshape(bm4 // 4, DTOT)
    o_ref[...] = (
        jnp.dot(x, w_ref[...], preferred_element_type=jnp.float32)
        + b_ref[...]
    )


def _mm_body_aliased(x_ref, w_ref, b_ref, _prev_ref, o_ref):
    _mm_body(x_ref, w_ref, b_ref, o_ref)


@functools.lru_cache(maxsize=None)
def _make_matmul(n_tok: int, bm: int, slice_blocks: int, k: int):
    base = k * slice_blocks
    in_specs = [
        pl.BlockSpec((bm * 4, 2 * EMB), lambda i: (i, 0)),
        pl.BlockSpec((DTOT, PROJ), lambda i: (0, 0)),
        pl.BlockSpec((1, PROJ), lambda i: (0, 0)),
    ]
    out_spec = pl.BlockSpec((bm, PROJ), lambda i: (i + base, 0))
    params = pltpu.CompilerParams(skip_device_barrier=True)
    if k == 0:
        return pl.pallas_call(
            _mm_body,
            grid=(slice_blocks,),
            in_specs=in_specs,
            out_specs=out_spec,
            out_shape=jax.ShapeDtypeStruct((n_tok, PROJ), jnp.float32),
            compiler_params=params,
        )
    return pl.pallas_call(
        _mm_body_aliased,
        grid=(slice_blocks,),
        in_specs=in_specs + [pl.BlockSpec(memory_space=pl.ANY)],
        out_specs=out_spec,
        out_shape=jax.ShapeDtypeStruct((n_tok, PROJ), jnp.float32),
        input_output_aliases={3: 0},
        compiler_params=params,
    )


NSLICE = 8
BM = 1600


def kernel(tokens, tables, proj_W, proj_b):
    B, T, F = tokens.shape
    n_tok = B * T
    n_rows = n_tok * F
    tok_s = n_tok // NSLICE
    rows_s = n_rows // NSLICE
    slice_blocks = tok_s // BM

    t128 = _make_transpose(2048)(tables.transpose(0, 2, 1))
    flat_tables = t128.reshape(F * VOCAB, EMB)

    v = tokens.astype(jnp.int32)
    vperm = ((v >> 11) << 11) | ((v & 1023) << 1) | ((v >> 10) & 1)
    offs = (jnp.arange(CHUNK, dtype=jnp.int32) % F) * VOCAB
    idx = vperm.reshape(n_rows // CHUNK, CHUNK) + offs

    wt = proj_W.T
    b2 = proj_b.reshape(1, PROJ)
    ichunks = rows_s // CHUNK

    out = None
    for k in range(NSLICE):
        g = _make_gather(rows_s, k * ichunks)(flat_tables, idx)
        xk = g.reshape(rows_s // 2, 2 * EMB)
        mm = _make_matmul(n_tok, BM, slice_blocks, k)
        out = mm(xk, wt, b2) if k == 0 else mm(xk, wt, b2, out)
    return out.reshape(B, T, PROJ)
